# baseline (speedup 1.0000x reference)
import functools

import jax
import jax.numpy as jnp
from jax import lax
from jax.experimental import pallas as pl
from jax.experimental.pallas import tpu as pltpu
from jax.experimental.pallas import tpu_sc as plsc

NN = 10000
EE = 320000
D = 128
DH = D // 2
NREL = 100
NP = 10240
K = 128
CH = 158
EPT = CH * K
EP = 16 * EPT
ROWS_PER_TILE = NP // 16
NB = 5
NI = 8
GA = 3
ILEAD = 5
CROWS_PER_TILE = NP * 4 // 16


def _scatter_pass_body(table, sdq, zeros, out, idxr, buf, acc,
                       sem_i, sem_g, sem_s):
    c = lax.axis_index("c")
    s = lax.axis_index("s")
    for q in range(2):
        pltpu.sync_copy(zeros.at[pl.ds(s * ROWS_PER_TILE, ROWS_PER_TILE)],
                        acc.at[q, pl.ds(s * ROWS_PER_TILE, ROWS_PER_TILE)])
    plsc.subcore_barrier()

    i_handles = {}
    g_handles = {}
    s_handles = {}
    half = table.at[c]
    my_sdq = sdq.at[s]

    def issue_idx(j):
        p = j % NI
        i_handles[j] = pltpu.async_copy(my_sdq.at[j], idxr.at[p],
                                        sem_i.at[p])

    def issue_gather(j):
        i_handles[j].wait()
        g_handles[j] = pltpu.async_copy(
            half.at[idxr.at[j % NI, 0]], buf.at[j % NB], sem_g.at[j % NB])

    for j in range(min(ILEAD, CH)):
        issue_idx(j)
    for j in range(min(GA, CH)):
        issue_gather(j)
    for j in range(CH):
        g_handles[j].wait()
        if j >= 2:
            s_handles[j - 2].wait()
        s_handles[j] = pltpu.async_copy(
            buf.at[j % NB], acc.at[j % 2].at[idxr.at[j % NI, 1]],
            sem_s.at[j % 2], add=True)
        if j + ILEAD < CH:
            issue_idx(j + ILEAD)
        if j + GA < CH:
            issue_gather(j + GA)
    for j in range(max(0, CH - 2), CH):
        s_handles[j].wait()

    plsc.subcore_barrier()
    for q in range(2):
        pltpu.sync_copy(acc.at[q, pl.ds(s * ROWS_PER_TILE, ROWS_PER_TILE)],
                        out.at[c, q, pl.ds(s * ROWS_PER_TILE, ROWS_PER_TILE)])


def _scatter_pass(table, sdq, zeros):
    mesh = plsc.VectorSubcoreMesh(core_axis_name="c", subcore_axis_name="s")
    k = functools.partial(
        pl.kernel,
        mesh=mesh,
        compiler_params=pltpu.CompilerParams(use_tc_tiling_on_sc=False),
        out_type=jax.ShapeDtypeStruct((2, 2, NP, DH), jnp.float32),
        scratch_types=[
            pltpu.VMEM((NI, 2, K), jnp.int32),
            pltpu.VMEM((NB, K, DH), jnp.float32),
            pltpu.VMEM_SHARED((2, NP, DH), jnp.float32),
            pltpu.SemaphoreType.DMA((NI,)),
            pltpu.SemaphoreType.DMA((NB,)),
            pltpu.SemaphoreType.DMA((2,)),
        ],
    )(_scatter_pass_body)
    return k(table, sdq, zeros)


def _count_pass_body(dstq, etyq, zeros, out, dst_v, ety_v, fr_v, buf, acc, sem):
    c = lax.axis_index("c")
    s = lax.axis_index("s")
    pltpu.sync_copy(zeros.at[pl.ds(s * CROWS_PER_TILE, CROWS_PER_TILE)],
                    acc.at[pl.ds(s * CROWS_PER_TILE, CROWS_PER_TILE)])
    pltpu.sync_copy(dstq.at[s], dst_v)
    pltpu.sync_copy(etyq.at[s], ety_v)

    def zrow(r, carry):
        z = jnp.zeros((16,), jnp.float32)
        buf[0, r] = z
        buf[1, r] = z
        return carry

    lax.fori_loop(0, K, zrow, 0)
    plsc.subcore_barrier()
    c64 = c * 64

    def onehot_ops(j, p, on):
        for k in range(8):
            sl = pl.ds(16 * k, 16)
            rl = ety_v[j, sl] - c64
            rlc = jnp.clip(rl, 0, 63)
            col = jnp.bitwise_and(rlc, 15)
            rowi = lax.iota(jnp.int32, 16) + 16 * k
            if on:
                inh = jnp.logical_and(rl >= 0, rl < 64)
                val = jnp.where(inh, 1.0, 0.0)
            else:
                val = jnp.zeros((16,), jnp.float32)
            plsc.store_scatter(buf.at[p], [rowi, col], val)

    def fill_fr(j, p):
        for k in range(8):
            sl = pl.ds(16 * k, 16)
            d16 = dst_v[j, sl]
            rlc = jnp.clip(ety_v[j, sl] - c64, 0, 63)
            fr_v[p, sl] = d16 * 4 + jnp.right_shift(rlc, 4)

    def body(i, carry):
        for p in range(2):
            j = 2 * i + p

            @pl.when(i > 0)
            def _():
                pltpu.make_async_copy(buf.at[p], acc.at[fr_v.at[p]],
                                      sem.at[p]).wait()
                onehot_ops(j - 2, p, False)

            fill_fr(j, p)
            onehot_ops(j, p, True)
            pltpu.async_copy(buf.at[p], acc.at[fr_v.at[p]], sem.at[p],
                             add=True)
        return carry

    lax.fori_loop(0, CH // 2, body, 0)
    for p in range(2):
        pltpu.make_async_copy(buf.at[p], acc.at[fr_v.at[p]], sem.at[p]).wait()
    plsc.subcore_barrier()
    pltpu.sync_copy(acc.at[pl.ds(s * CROWS_PER_TILE, CROWS_PER_TILE)],
                    out.at[c, pl.ds(s * CROWS_PER_TILE, CROWS_PER_TILE)])


def _count_pass(dstq, etyq, zeros):
    mesh = plsc.VectorSubcoreMesh(core_axis_name="c", subcore_axis_name="s")
    k = functools.partial(
        pl.kernel,
        mesh=mesh,
        compiler_params=pltpu.CompilerParams(use_tc_tiling_on_sc=False,
                                             needs_layout_passes=False),
        out_type=jax.ShapeDtypeStruct((2, NP * 4, 16), jnp.float32),
        scratch_types=[
            pltpu.VMEM((CH, K), jnp.int32),
            pltpu.VMEM((CH, K), jnp.int32),
            pltpu.VMEM((2, K), jnp.int32),
            pltpu.VMEM((2, K, 16), jnp.float32),
            pltpu.VMEM_SHARED((NP * 4, 16), jnp.float32),
            pltpu.SemaphoreType.DMA((2,)),
        ],
    )(_count_pass_body)
    return k(dstq, etyq, zeros)


def _rel_chain_body(relp_ref, wr0_ref, wr1_ref, r1_ref, r2_ref):
    r1 = jnp.dot(relp_ref[...], wr0_ref[...], preferred_element_type=jnp.float32)
    r1_ref[...] = r1
    r2_ref[...] = jnp.dot(r1, wr1_ref[...], preferred_element_type=jnp.float32)


def _rel_chain(relp, wr0, wr1):
    return pl.pallas_call(
        _rel_chain_body,
        out_shape=[jax.ShapeDtypeStruct((D, D), jnp.float32),
                   jax.ShapeDtypeStruct((D, D), jnp.float32)],
    )(relp, wr0, wr1)


def _layer_body(g_ref, c_ref, h_ref, rel_ref, w_ref, wl_ref, o_ref, *, act,
                split_out):
    csum = jnp.concatenate([c_ref[0], c_ref[1]], axis=1)
    deg = jnp.sum(csum, axis=1)
    norm = 1.0 / jnp.maximum(deg, 1.0)
    g = jnp.concatenate([g_ref[0, 0] + g_ref[0, 1],
                         g_ref[1, 0] + g_ref[1, 1]], axis=1)
    h = jnp.concatenate([h_ref[0], h_ref[1]], axis=1)
    agg = g - jnp.dot(csum, rel_ref[...], preferred_element_type=jnp.float32)
    hn = (jnp.dot(agg * norm[:, None], w_ref[...],
                  preferred_element_type=jnp.float32)
          + jnp.dot(h, wl_ref[...], preferred_element_type=jnp.float32))
    if act:
        hn = jnp.maximum(hn, 0.0)
    if split_out:
        o_ref[0] = hn[:, :DH]
        o_ref[1] = hn[:, DH:]
    else:
        o_ref[...] = hn


def _layer_tc(Gp, Cp, h, rel, W, Wl, act, split_out):
    blk = 1024
    grid = (NP // blk,)
    g_spec = pl.BlockSpec((2, 2, blk, DH), lambda i: (0, 0, i, 0))
    half_spec = pl.BlockSpec((2, blk, DH), lambda i: (0, i, 0))
    mat_spec = pl.BlockSpec((D, D), lambda i: (0, 0))
    if split_out:
        out_spec = half_spec
        out_shape = jax.ShapeDtypeStruct((2, NP, DH), jnp.float32)
    else:
        out_spec = pl.BlockSpec((blk, D), lambda i: (i, 0))
        out_shape = jax.ShapeDtypeStruct((NP, D), jnp.float32)
    return pl.pallas_call(
        functools.partial(_layer_body, act=act, split_out=split_out),
        grid=grid,
        in_specs=[g_spec, half_spec, half_spec, mat_spec, mat_spec,
                  mat_spec],
        out_specs=out_spec,
        out_shape=out_shape,
    )(Gp, Cp, h, rel, W, Wl)


def kernel(features, relations, edge_index, edge_type, Ws, Wls, Wrs):
    src = edge_index[0]
    dst = edge_index[1]
    pad = EP - EE
    srcq = jnp.concatenate(
        [src, jnp.zeros((pad,), jnp.int32)]).reshape(16, CH, K)
    dstq = jnp.concatenate(
        [dst, jnp.full((pad,), NN, jnp.int32)]).reshape(16, CH, K)
    etyq = jnp.concatenate(
        [edge_type, jnp.zeros((pad,), jnp.int32)]).reshape(16, CH, K)
    zeros = jnp.zeros((NP, DH), jnp.float32)
    zeros_c = jnp.zeros((NP * 4, 16), jnp.float32)
    hpad = jnp.pad(features, ((0, NP - NN), (0, 0)))
    h = jnp.stack([hpad[:, :DH], hpad[:, DH:]])
    relp = jnp.pad(relations, ((0, D - NREL), (0, 0)))

    sdq = jnp.stack([srcq, dstq], axis=2)
    Cp = _count_pass(dstq, etyq, zeros_c).reshape(2, NP, DH)
    rel1, rel2 = _rel_chain(relp, Wrs[0], Wrs[1])
    rels = (relp, rel1, rel2)
    for l in range(3):
        Gp = _scatter_pass(h, sdq, zeros)
        h = _layer_tc(Gp, Cp, h, rels[l], Ws[l], Wls[l], act=(l < 2),
                      split_out=(l < 2))
    return h[:NN]


# speedup vs baseline: 1.0922x; 1.0922x over previous
import functools

import jax
import jax.numpy as jnp
from jax import lax
from jax.experimental import pallas as pl
from jax.experimental.pallas import tpu as pltpu
from jax.experimental.pallas import tpu_sc as plsc

NN = 10000
EE = 320000
D = 128
DH = D // 2
NREL = 100
NP = 10240
K = 128
CH = 158
EPT = CH * K
EP = 16 * EPT
ROWS_PER_TILE = NP // 16
NBUF = 6
GAHEAD = 3
CROWS_PER_TILE = NP * 4 // 16


def _scatter_pass_body(table, srcq, dstq, zeros, out, src_v, dst_v, buf,
                       acc, sem_g, sem_s):
    c = lax.axis_index("c")
    s = lax.axis_index("s")
    pltpu.sync_copy(zeros.at[pl.ds(s * ROWS_PER_TILE, ROWS_PER_TILE)],
                    acc.at[pl.ds(s * ROWS_PER_TILE, ROWS_PER_TILE)])
    pltpu.sync_copy(srcq.at[s], src_v)
    pltpu.sync_copy(dstq.at[s], dst_v)
    plsc.subcore_barrier()

    g_handles = {}
    s_handles = {}
    half = table.at[c]

    def issue_gather(j):
        b = j % NBUF
        g_handles[j] = pltpu.async_copy(
            half.at[src_v.at[j]], buf.at[b], sem_g.at[b])

    for j in range(GAHEAD):
        issue_gather(j)
    for j in range(CH):
        b = j % NBUF
        g_handles[j].wait()
        if j > 0:
            s_handles[j - 1].wait()
        s_handles[j] = pltpu.async_copy(
            buf.at[b], acc.at[dst_v.at[j]], sem_s.at[b], add=True)
        jn = j + GAHEAD
        if jn < CH:
            issue_gather(jn)
    s_handles[CH - 1].wait()

    plsc.subcore_barrier()
    pltpu.sync_copy(acc.at[pl.ds(s * ROWS_PER_TILE, ROWS_PER_TILE)],
                    out.at[c, pl.ds(s * ROWS_PER_TILE, ROWS_PER_TILE)])


def _scatter_pass(table, srcq, dstq, zeros):
    mesh = plsc.VectorSubcoreMesh(core_axis_name="c", subcore_axis_name="s")
    k = functools.partial(
        pl.kernel,
        mesh=mesh,
        compiler_params=pltpu.CompilerParams(use_tc_tiling_on_sc=False),
        out_type=jax.ShapeDtypeStruct((2, NP, DH), jnp.float32),
        scratch_types=[
            pltpu.VMEM((CH, K), jnp.int32),
            pltpu.VMEM((CH, K), jnp.int32),
            pltpu.VMEM((NBUF, K, DH), jnp.float32),
            pltpu.VMEM_SHARED((NP, DH), jnp.float32),
            pltpu.SemaphoreType.DMA((NBUF,)),
            pltpu.SemaphoreType.DMA((NBUF,)),
        ],
    )(_scatter_pass_body)
    return k(table, srcq, dstq, zeros)


def _count_pass_body(dstq, etyq, zeros, out, dst_v, ety_v, fr_v, buf, acc, sem):
    c = lax.axis_index("c")
    s = lax.axis_index("s")
    pltpu.sync_copy(zeros.at[pl.ds(s * CROWS_PER_TILE, CROWS_PER_TILE)],
                    acc.at[pl.ds(s * CROWS_PER_TILE, CROWS_PER_TILE)])
    pltpu.sync_copy(dstq.at[s], dst_v)
    pltpu.sync_copy(etyq.at[s], ety_v)

    def zrow(r, carry):
        z = jnp.zeros((16,), jnp.float32)
        buf[0, r] = z
        buf[1, r] = z
        return carry

    lax.fori_loop(0, K, zrow, 0)
    plsc.subcore_barrier()
    c64 = c * 64

    def onehot_ops(j, p, on):
        for k in range(8):
            sl = pl.ds(16 * k, 16)
            rl = ety_v[j, sl] - c64
            rlc = jnp.clip(rl, 0, 63)
            col = jnp.bitwise_and(rlc, 15)
            rowi = lax.iota(jnp.int32, 16) + 16 * k
            if on:
                inh = jnp.logical_and(rl >= 0, rl < 64)
                val = jnp.where(inh, 1.0, 0.0)
            else:
                val = jnp.zeros((16,), jnp.float32)
            plsc.store_scatter(buf.at[p], [rowi, col], val)

    def fill_fr(j, p):
        for k in range(8):
            sl = pl.ds(16 * k, 16)
            d16 = dst_v[j, sl]
            rlc = jnp.clip(ety_v[j, sl] - c64, 0, 63)
            fr_v[p, sl] = d16 * 4 + jnp.right_shift(rlc, 4)

    def body(i, carry):
        for p in range(2):
            j = 2 * i + p

            @pl.when(i > 0)
            def _():
                pltpu.make_async_copy(buf.at[p], acc.at[fr_v.at[p]],
                                      sem.at[p]).wait()
                onehot_ops(j - 2, p, False)

            fill_fr(j, p)
            onehot_ops(j, p, True)
            pltpu.async_copy(buf.at[p], acc.at[fr_v.at[p]], sem.at[p],
                             add=True)
        return carry

    lax.fori_loop(0, CH // 2, body, 0)
    for p in range(2):
        pltpu.make_async_copy(buf.at[p], acc.at[fr_v.at[p]], sem.at[p]).wait()
    plsc.subcore_barrier()
    pltpu.sync_copy(acc.at[pl.ds(s * CROWS_PER_TILE, CROWS_PER_TILE)],
                    out.at[c, pl.ds(s * CROWS_PER_TILE, CROWS_PER_TILE)])


def _count_pass(dstq, etyq, zeros):
    mesh = plsc.VectorSubcoreMesh(core_axis_name="c", subcore_axis_name="s")
    k = functools.partial(
        pl.kernel,
        mesh=mesh,
        compiler_params=pltpu.CompilerParams(use_tc_tiling_on_sc=False,
                                             needs_layout_passes=False),
        out_type=jax.ShapeDtypeStruct((2, NP * 4, 16), jnp.float32),
        scratch_types=[
            pltpu.VMEM((CH, K), jnp.int32),
            pltpu.VMEM((CH, K), jnp.int32),
            pltpu.VMEM((2, K), jnp.int32),
            pltpu.VMEM((2, K, 16), jnp.float32),
            pltpu.VMEM_SHARED((NP * 4, 16), jnp.float32),
            pltpu.SemaphoreType.DMA((2,)),
        ],
    )(_count_pass_body)
    return k(dstq, etyq, zeros)


def _rel_chain_body(relp_ref, wr0_ref, wr1_ref, r1_ref, r2_ref):
    r1 = jnp.dot(relp_ref[...], wr0_ref[...], preferred_element_type=jnp.float32)
    r1_ref[...] = r1
    r2_ref[...] = jnp.dot(r1, wr1_ref[...], preferred_element_type=jnp.float32)


def _rel_chain(relp, wr0, wr1):
    return pl.pallas_call(
        _rel_chain_body,
        out_shape=[jax.ShapeDtypeStruct((D, D), jnp.float32),
                   jax.ShapeDtypeStruct((D, D), jnp.float32)],
    )(relp, wr0, wr1)


def _layer_body(g_ref, c_ref, h_ref, rel_ref, w_ref, wl_ref, o_ref, *, act,
                split_out):
    csum = jnp.concatenate([c_ref[0], c_ref[1]], axis=1)
    deg = jnp.sum(csum, axis=1)
    norm = 1.0 / jnp.maximum(deg, 1.0)
    g = jnp.concatenate([g_ref[0], g_ref[1]], axis=1)
    h = jnp.concatenate([h_ref[0], h_ref[1]], axis=1)
    agg = g - jnp.dot(csum, rel_ref[...], preferred_element_type=jnp.float32)
    hn = (jnp.dot(agg * norm[:, None], w_ref[...],
                  preferred_element_type=jnp.float32)
          + jnp.dot(h, wl_ref[...], preferred_element_type=jnp.float32))
    if act:
        hn = jnp.maximum(hn, 0.0)
    if split_out:
        o_ref[0] = hn[:, :DH]
        o_ref[1] = hn[:, DH:]
    else:
        o_ref[...] = hn


def _layer_tc(Gp, Cp, h, rel, W, Wl, act, split_out):
    blk = 1024
    grid = (NP // blk,)
    half_spec = pl.BlockSpec((2, blk, DH), lambda i: (0, i, 0))
    g_spec = half_spec
    mat_spec = pl.BlockSpec((D, D), lambda i: (0, 0))
    if split_out:
        out_spec = half_spec
        out_shape = jax.ShapeDtypeStruct((2, NP, DH), jnp.float32)
    else:
        out_spec = pl.BlockSpec((blk, D), lambda i: (i, 0))
        out_shape = jax.ShapeDtypeStruct((NP, D), jnp.float32)
    return pl.pallas_call(
        functools.partial(_layer_body, act=act, split_out=split_out),
        grid=grid,
        in_specs=[g_spec, half_spec, half_spec, mat_spec, mat_spec,
                  mat_spec],
        out_specs=out_spec,
        out_shape=out_shape,
    )(Gp, Cp, h, rel, W, Wl)


def kernel(features, relations, edge_index, edge_type, Ws, Wls, Wrs):
    src = edge_index[0]
    dst = edge_index[1]
    pad = EP - EE
    srcq = jnp.concatenate(
        [src, jnp.zeros((pad,), jnp.int32)]).reshape(16, CH, K)
    dstq = jnp.concatenate(
        [dst, jnp.full((pad,), NN, jnp.int32)]).reshape(16, CH, K)
    etyq = jnp.concatenate(
        [edge_type, jnp.zeros((pad,), jnp.int32)]).reshape(16, CH, K)
    zeros = jnp.zeros((NP, DH), jnp.float32)
    zeros_c = jnp.zeros((NP * 4, 16), jnp.float32)
    hpad = jnp.pad(features, ((0, NP - NN), (0, 0)))
    h = jnp.stack([hpad[:, :DH], hpad[:, DH:]])
    relp = jnp.pad(relations, ((0, D - NREL), (0, 0)))

    Cp = _count_pass(dstq, etyq, zeros_c).reshape(2, NP, DH)
    rel1, rel2 = _rel_chain(relp, Wrs[0], Wrs[1])
    rels = (relp, rel1, rel2)
    for l in range(3):
        Gp = _scatter_pass(h, srcq, dstq, zeros)
        h = _layer_tc(Gp, Cp, h, rels[l], Ws[l], Wls[l], act=(l < 2),
                      split_out=(l < 2))
    return h[:NN]
